# baseline (speedup 1.0000x reference)
import functools

import jax
import jax.numpy as jnp
import numpy as np
from jax import lax
from jax.experimental import pallas as pl
from jax.experimental.pallas import tpu as pltpu
from jax.experimental.pallas import tpu_sc as plsc

ROWS, COLS = 128, 32768
N = ROWS * COLS
NC, NS = 2, 16
NW = NC * NS
PER_W = N // NW
CHUNK = 16384
NCHUNK = PER_W // CHUNK
ROWCH = COLS // CHUNK
VECS = CHUNK // 16

B1 = 1024
SHIFT1 = 22
U1 = 4
H1 = U1 * 16 * B1

B2 = 512
SHIFT2 = 13
U2 = 2
NT = 4
REG = 16 * B2
H2 = (NT + 1) * U2 * REG

_SIGN = np.uint32(0x80000000)
_MININT = np.int32(-0x80000000)


def _mono_key(v):
    ki = plsc.bitcast(v, jnp.int32)
    flip = (ki >> 31) | _MININT
    return plsc.bitcast(ki ^ flip, jnp.uint32)


def _zero(ref, nwords):
    z = jnp.zeros((16,), jnp.int32)

    @plsc.parallel_loop(0, nwords // 16)
    def _(i):
        ref[pl.ds(i * 16, 16)] = z


def _fold_rows(hist, base, nrows, width):
    @plsc.parallel_loop(0, width // 16)
    def _(j):
        off = base + j * 16
        acc = hist[pl.ds(off, 16)]
        for r in range(1, nrows):
            acc = acc + hist[pl.ds(off + r * width, 16)]
        hist[pl.ds(off, 16)] = acc


def _stream_chunks(x_hbm, buf, sems, wid, unroll, compute_vec):
    def copy_in(ci, b):
        row = wid * (NCHUNK // ROWCH) + ci // ROWCH
        col = (ci % ROWCH) * CHUNK
        return pltpu.async_copy(
            x_hbm.at[row, pl.ds(col, CHUNK)], buf.at[b], sems[b])

    handles = [copy_in(0, 0), copy_in(1, 1)]
    for ci in range(NCHUNK):
        b = ci % 2
        handles[b].wait()

        @plsc.parallel_loop(0, VECS // unroll, unroll=8 // unroll)
        def _(i, _b=b):
            for u in range(unroll):
                compute_vec(u, buf[_b, pl.ds((i * unroll + u) * 16, 16)])

        if ci + 2 < NCHUNK:
            handles[b] = copy_in(ci + 2, b)


def _pass1_body(x_hbm, out_hbm, hist, buf, fbuf, sem0, sem1):
    wid = lax.axis_index("s") * NC + lax.axis_index("c")
    lane_base = lax.iota(jnp.int32, 16) * B1
    ones = jnp.ones((16,), jnp.int32)
    _zero(hist, H1)

    def compute_vec(u, v):
        key = _mono_key(v)
        bucket = plsc.bitcast(key >> np.uint32(SHIFT1), jnp.int32)
        plsc.addupdate_scatter(hist, [(lane_base + u * 16 * B1) + bucket],
                               ones)

    _stream_chunks(x_hbm, buf, (sem0, sem1), wid, U1, compute_vec)
    _fold_rows(hist, 0, U1 * 16, B1)

    @plsc.parallel_loop(0, B1 // 16)
    def _(j):
        fbuf[pl.ds(j * 16, 16)] = plsc.bitcast(hist[pl.ds(j * 16, 16)],
                                               jnp.float32)

    pltpu.sync_copy(fbuf, out_hbm.at[pl.ds(wid * B1, B1)])


def _pass2_body(x_hbm, h1_hbm, out_hbm, hist, buf, lut, sem0, sem1):
    wid = lax.axis_index("s") * NC + lax.axis_index("c")
    lane = lax.iota(jnp.int32, 16)
    lane_base = lane * B2
    ones = jnp.ones((16,), jnp.int32)
    _zero(hist, NT * U2 * REG)

    pltpu.sync_copy(h1_hbm.at[pl.ds(0, CHUNK)], buf.at[0])
    pltpu.sync_copy(h1_hbm.at[pl.ds(CHUNK, CHUNK)], buf.at[1])

    @plsc.parallel_loop(0, B1 // 16)
    def _(j):
        acc = plsc.bitcast(buf[0, pl.ds(j * 16, 16)], jnp.int32)
        for r in range(1, NW):
            acc = acc + plsc.bitcast(
                buf[r // 16, pl.ds((r % 16) * B1 + j * 16, 16)], jnp.int32)
        lut[pl.ds(j * 16, 16)] = acc

    def cs_body(j, carry):
        v = lut[pl.ds(j * 16, 16)]
        s = plsc.cumsum(v) + jnp.broadcast_to(carry, (16,))
        lut[pl.ds(j * 16, 16)] = s
        return carry + jnp.sum(v)

    lax.fori_loop(0, B1 // 16, cs_body, jnp.int32(0))

    kvecs = [jnp.full((16,), k, jnp.int32) for k in _RANKS]

    def rf_body(j, ps):
        c = lut[pl.ds(j * 16, 16)]
        return tuple(
            p + plsc.all_reduce_population_count(c <= kv)
            for p, kv in zip(ps, kvecs))

    zero16 = jnp.zeros((16,), jnp.int32)
    prefs = lax.fori_loop(0, B1 // 16, rf_body,
                          (zero16, zero16, zero16, zero16))

    trash = jnp.full((16,), NT * U2 * REG, jnp.int32)

    @plsc.parallel_loop(0, B1 // 16)
    def _(i):
        lut[pl.ds(i * 16, 16)] = trash

    lane0 = lane == 0
    for t in range(NT - 1, -1, -1):
        plsc.store_scatter(lut, [prefs[t]],
                           jnp.full((16,), t * U2 * REG, jnp.int32),
                           mask=lane0)

    def compute_vec(u, v):
        key = _mono_key(v)
        hi = plsc.bitcast(key >> np.uint32(SHIFT1), jnp.int32)
        bucket = plsc.bitcast(
            (key >> np.uint32(SHIFT2)) & np.uint32(B2 - 1), jnp.int32)
        base = plsc.load_gather(lut, [hi])
        plsc.addupdate_scatter(
            hist, [base + ((lane_base + u * REG) + bucket)], ones)

    _stream_chunks(x_hbm, buf, (sem0, sem1), wid, U2, compute_vec)
    for t in range(NT):
        _fold_rows(hist, t * U2 * REG, U2 * 16, B2)
        pltpu.sync_copy(hist.at[pl.ds(t * U2 * REG, B2)],
                        out_hbm.at[pl.ds((wid * NT + t) * B2, B2)])


_mesh = plsc.VectorSubcoreMesh(core_axis_name="c", subcore_axis_name="s")
_params = pltpu.CompilerParams(needs_layout_passes=False)

_pass1 = functools.partial(
    pl.kernel,
    out_type=jax.ShapeDtypeStruct((NW * B1,), jnp.float32),
    scratch_types=[
        pltpu.VMEM((H1,), jnp.int32),
        pltpu.VMEM((2, CHUNK), jnp.float32),
        pltpu.VMEM((B1,), jnp.float32),
        pltpu.SemaphoreType.DMA,
        pltpu.SemaphoreType.DMA,
    ],
    mesh=_mesh,
    compiler_params=_params,
)(_pass1_body)

_pass2 = functools.partial(
    pl.kernel,
    out_type=jax.ShapeDtypeStruct((NW * NT * B2,), jnp.int32),
    scratch_types=[
        pltpu.VMEM((H2,), jnp.int32),
        pltpu.VMEM((2, CHUNK), jnp.float32),
        pltpu.VMEM((B1,), jnp.int32),
        pltpu.SemaphoreType.DMA,
        pltpu.SemaphoreType.DMA,
    ],
    mesh=_mesh,
    compiler_params=_params,
)(_pass2_body)

_POS_LO = 0.05 * (N - 1)
_POS_HI = 0.95 * (N - 1)
_K_LO = int(_POS_LO)
_K_HI = int(_POS_HI)
_F_LO = _POS_LO - _K_LO
_F_HI = _POS_HI - _K_HI
_RANKS = (_K_LO, _K_LO + 1, _K_HI, _K_HI + 1)


def kernel(x, low, high):
    x = lax.stop_gradient(x)

    out1 = _pass1(x)
    hist1 = lax.bitcast_convert_type(out1, jnp.int32).reshape(NW, B1).sum(
        axis=0)
    c1 = jnp.cumsum(hist1)
    ranks = jnp.array(_RANKS, jnp.int32)
    p1 = jnp.sum(c1[None, :] <= ranks[:, None], axis=1).astype(jnp.int32)
    below = jnp.where(p1 > 0, c1[jnp.maximum(p1 - 1, 0)], 0)
    r = ranks - below

    out2 = _pass2(x, out1)
    hist2 = out2.reshape(NW, NT, B2).sum(axis=0)
    first = jnp.argmax(p1[None, :] == p1[:, None], axis=1)
    hist_eff = hist2[first]
    c2 = jnp.cumsum(hist_eff, axis=1)
    b2 = jnp.sum(c2 <= r[:, None], axis=1).astype(jnp.uint32)

    key_mid = ((((p1.astype(jnp.uint32) << 9) | b2) << 13)
               | jnp.uint32(1 << 12))
    orig = jnp.where(key_mid >= _SIGN, key_mid ^ _SIGN, ~key_mid)
    vals = lax.bitcast_convert_type(orig, jnp.float32)

    q_lo = vals[0] + jnp.float32(_F_LO) * (vals[1] - vals[0])
    q_hi = vals[2] + jnp.float32(_F_HI) * (vals[3] - vals[2])

    decay = jnp.float32(0.99)
    new_low = decay * low + (1.0 - decay) * q_lo
    new_high = decay * high + (1.0 - decay) * q_hi
    inverse_scale = jnp.maximum(jnp.float32(1.0), new_high - new_low)
    return (new_low, inverse_scale)


# speedup vs baseline: 1.2809x; 1.2809x over previous
import functools

import jax
import jax.numpy as jnp
import numpy as np
from jax import lax
from jax.experimental import pallas as pl
from jax.experimental.pallas import tpu as pltpu
from jax.experimental.pallas import tpu_sc as plsc

ROWS, COLS = 128, 32768
N = ROWS * COLS
NC, NS = 2, 16
NW = NC * NS
PER_W = N // NW
CHUNK = 32768
NCHUNK = PER_W // CHUNK
ROWCH = COLS // CHUNK
VECS = CHUNK // 16

B1 = 1024
SHIFT1 = 22
U1 = 2
H1 = U1 * 16 * B1

B2 = 256
SHIFT2 = 14
U2 = 2
NT = 4
REG = 16 * B2
H2 = (NT + 1) * U2 * REG

_SIGN = np.uint32(0x80000000)
_MININT = np.int32(-0x80000000)


def _mono_key(v):
    ki = plsc.bitcast(v, jnp.int32)
    flip = (ki >> 31) | _MININT
    return plsc.bitcast(ki ^ flip, jnp.uint32)


def _zero(ref, nwords):
    z = jnp.zeros((16,), jnp.int32)

    @plsc.parallel_loop(0, nwords // 16)
    def _(i):
        ref[pl.ds(i * 16, 16)] = z


def _fold_rows(hist, base, nrows, width):
    @plsc.parallel_loop(0, width // 16)
    def _(j):
        off = base + j * 16
        acc = hist[pl.ds(off, 16)]
        for r in range(1, nrows):
            acc = acc + hist[pl.ds(off + r * width, 16)]
        hist[pl.ds(off, 16)] = acc


def _stream_chunks(x_hbm, buf, sems, wid, unroll, compute_vec, prelude):
    def copy_in(ci, b):
        row = wid * (NCHUNK // ROWCH) + ci // ROWCH
        col = (ci % ROWCH) * CHUNK
        return pltpu.async_copy(
            x_hbm.at[row, pl.ds(col, CHUNK)], buf.at[b], sems[b])

    handles = [copy_in(0, 0), copy_in(1, 1)]
    prelude()
    for ci in range(NCHUNK):
        b = ci % 2
        handles[b].wait()

        @plsc.parallel_loop(0, VECS // unroll, unroll=8 // unroll)
        def _(i, _b=b):
            for u in range(unroll):
                compute_vec(u, buf[_b, pl.ds((i * unroll + u) * 16, 16)])

        if ci + 2 < NCHUNK:
            handles[b] = copy_in(ci + 2, b)


def _pass1_body(x_hbm, out_hbm, hist, buf, sem0, sem1):
    wid = lax.axis_index("s") * NC + lax.axis_index("c")
    lane_base = lax.iota(jnp.int32, 16) * B1
    ones = jnp.ones((16,), jnp.int32)

    def compute_vec(u, v):
        key = _mono_key(v)
        bucket = plsc.bitcast(key >> np.uint32(SHIFT1), jnp.int32)
        plsc.addupdate_scatter(hist, [(lane_base + u * 16 * B1) + bucket],
                               ones)

    _stream_chunks(x_hbm, buf, (sem0, sem1), wid, U1, compute_vec,
                   prelude=lambda: _zero(hist, H1))
    _fold_rows(hist, 0, U1 * 16, B1)
    pltpu.sync_copy(hist.at[pl.ds(0, B1)], out_hbm.at[pl.ds(wid * B1, B1)])


def _pass2_body(x_hbm, pref_hbm, out_hbm, hist, buf, lut, pbuf, sem0, sem1):
    wid = lax.axis_index("s") * NC + lax.axis_index("c")
    lane = lax.iota(jnp.int32, 16)
    lane_base = lane * B2
    ones = jnp.ones((16,), jnp.int32)

    def prelude():
        _zero(hist, NT * U2 * REG)
        pltpu.sync_copy(pref_hbm, pbuf)

        trash = jnp.full((16,), NT * U2 * REG, jnp.int32)

        @plsc.parallel_loop(0, B1 // 16)
        def _(i):
            lut[pl.ds(i * 16, 16)] = trash

        lane0 = lane == 0
        for t in range(NT - 1, -1, -1):
            plsc.store_scatter(lut, [pbuf[t]],
                               jnp.full((16,), t * U2 * REG, jnp.int32),
                               mask=lane0)

    def compute_vec(u, v):
        key = _mono_key(v)
        hi = plsc.bitcast(key >> np.uint32(SHIFT1), jnp.int32)
        bucket = plsc.bitcast(
            (key >> np.uint32(SHIFT2)) & np.uint32(B2 - 1), jnp.int32)
        base = plsc.load_gather(lut, [hi])
        plsc.addupdate_scatter(
            hist, [base + ((lane_base + u * REG) + bucket)], ones)

    _stream_chunks(x_hbm, buf, (sem0, sem1), wid, U2, compute_vec, prelude)
    for t in range(NT):
        _fold_rows(hist, t * U2 * REG, U2 * 16, B2)
        pltpu.sync_copy(hist.at[pl.ds(t * U2 * REG, B2)],
                        out_hbm.at[pl.ds((wid * NT + t) * B2, B2)])


_mesh = plsc.VectorSubcoreMesh(core_axis_name="c", subcore_axis_name="s")
_params = pltpu.CompilerParams(needs_layout_passes=False)

_pass1 = functools.partial(
    pl.kernel,
    out_type=jax.ShapeDtypeStruct((NW * B1,), jnp.int32),
    scratch_types=[
        pltpu.VMEM((H1,), jnp.int32),
        pltpu.VMEM((2, CHUNK), jnp.float32),
        pltpu.SemaphoreType.DMA,
        pltpu.SemaphoreType.DMA,
    ],
    mesh=_mesh,
    compiler_params=_params,
)(_pass1_body)

_pass2 = functools.partial(
    pl.kernel,
    out_type=jax.ShapeDtypeStruct((NW * NT * B2,), jnp.int32),
    scratch_types=[
        pltpu.VMEM((H2,), jnp.int32),
        pltpu.VMEM((2, CHUNK), jnp.float32),
        pltpu.VMEM((B1,), jnp.int32),
        pltpu.VMEM((NT, 16), jnp.int32),
        pltpu.SemaphoreType.DMA,
        pltpu.SemaphoreType.DMA,
    ],
    mesh=_mesh,
    compiler_params=_params,
)(_pass2_body)

_POS_LO = 0.05 * (N - 1)
_POS_HI = 0.95 * (N - 1)
_K_LO = int(_POS_LO)
_K_HI = int(_POS_HI)
_F_LO = _POS_LO - _K_LO
_F_HI = _POS_HI - _K_HI
_RANKS = (_K_LO, _K_LO + 1, _K_HI, _K_HI + 1)


def kernel(x, low, high):
    x = lax.stop_gradient(x)

    out1 = _pass1(x)
    hist1 = out1.reshape(NW, B1).sum(axis=0)
    c1 = jnp.cumsum(hist1)
    ranks = jnp.array(_RANKS, jnp.int32)
    p1 = jnp.sum(c1[None, :] <= ranks[:, None], axis=1).astype(jnp.int32)
    below = jnp.where(p1 > 0, c1[jnp.maximum(p1 - 1, 0)], 0)
    r = ranks - below

    prefs = jnp.broadcast_to(p1[:, None], (NT, 16)).astype(jnp.int32)
    out2 = _pass2(x, prefs)
    hist2 = out2.reshape(NW, NT, B2).sum(axis=0)
    first = jnp.argmax(p1[None, :] == p1[:, None], axis=1)
    hist_eff = hist2[first]
    c2 = jnp.cumsum(hist_eff, axis=1)
    b2 = jnp.sum(c2 <= r[:, None], axis=1).astype(jnp.uint32)

    key_mid = ((((p1.astype(jnp.uint32) << 8) | b2) << 14)
               | jnp.uint32(1 << 13))
    orig = jnp.where(key_mid >= _SIGN, key_mid ^ _SIGN, ~key_mid)
    vals = lax.bitcast_convert_type(orig, jnp.float32)

    q_lo = vals[0] + jnp.float32(_F_LO) * (vals[1] - vals[0])
    q_hi = vals[2] + jnp.float32(_F_HI) * (vals[3] - vals[2])

    decay = jnp.float32(0.99)
    new_low = decay * low + (1.0 - decay) * q_lo
    new_high = decay * high + (1.0 - decay) * q_hi
    inverse_scale = jnp.maximum(jnp.float32(1.0), new_high - new_low)
    return (new_low, inverse_scale)
